# Initial kernel scaffold; baseline (speedup 1.0000x reference)
import jax
import jax.numpy as jnp
from jax.experimental import pallas as pl


def kernel(x, hyperedge_index, batch, params):
    raise NotImplementedError("write your pallas kernel here")


# speedup vs baseline: 11.8348x; 11.8348x over previous
import functools

import jax
import jax.numpy as jnp
from jax import lax
from jax.experimental import pallas as pl
from jax.experimental.pallas import tpu as pltpu
from jax.experimental.pallas import tpu_sc as plsc

N = 10000
NP = 10240
D = 128
E_INC = 320000
NC, NS = 2, 16
NW = NC * NS
PER_TILE = E_INC // NW
CHUNK = 80
NCH = PER_TILE // CHUNK
ROWS_PT = NP // NS
ZROWS = 16
DEG_D = 16

_MESH = plsc.VectorSubcoreMesh(
    core_axis_name="c", subcore_axis_name="s", num_cores=NC, num_subcores=NS
)

_f32 = jnp.float32


@functools.partial(
    pl.kernel,
    out_type=jax.ShapeDtypeStruct((NC, NP, D), _f32),
    mesh=_MESH,
    scratch_types=[
        pltpu.VMEM_SHARED((NP, D), _f32),
        pltpu.VMEM((NCH, CHUNK), jnp.int32),
        pltpu.VMEM((CHUNK, D), _f32),
        pltpu.VMEM((ZROWS, D), _f32),
        pltpu.SemaphoreType.DMA,
    ],
)
def _sc_count(ones_h, dst_h, out_h, acc, dst_v, rows_v, zero_v, sem):
    c = lax.axis_index("c")
    s = lax.axis_index("s")
    wid = c * NS + s

    zero16 = jnp.zeros((16,), _f32)

    def fill_zero(t, _):
        i = t // (D // 16)
        j = t % (D // 16)
        zero_v[i, pl.ds(j * 16, 16)] = zero16
        return 0

    lax.fori_loop(0, ZROWS * (D // 16), fill_zero, 0)

    base_r = s * ROWS_PT

    def zcopy(r, _):
        pltpu.sync_copy(zero_v, acc.at[pl.ds(base_r + r * ZROWS, ZROWS)])
        return 0

    lax.fori_loop(0, ROWS_PT // ZROWS, zcopy, 0)
    plsc.subcore_barrier()

    pltpu.sync_copy(dst_h.at[wid], dst_v)
    pltpu.async_copy(ones_h.at[pl.ds(0, CHUNK)], rows_v, sem).wait()

    def body(j, _):
        pltpu.sync_copy(rows_v, acc.at[dst_v.at[j]], add=True)
        return 0

    lax.fori_loop(0, NCH, body, 0)
    plsc.subcore_barrier()

    pltpu.sync_copy(acc.at[pl.ds(base_r, ROWS_PT)],
                    out_h.at[c, pl.ds(base_r, ROWS_PT)])


@functools.partial(
    pl.kernel,
    out_type=jax.ShapeDtypeStruct((NC, NP, D), _f32),
    mesh=_MESH,
    scratch_types=[
        pltpu.VMEM_SHARED((NP, D), _f32),
        pltpu.VMEM((NCH, CHUNK), jnp.int32),
        pltpu.VMEM((NCH, CHUNK), jnp.int32),
        pltpu.VMEM((CHUNK, D), _f32),
        pltpu.VMEM((ZROWS, D), _f32),
        pltpu.SemaphoreType.DMA,
    ],
)
def _sc_phase(table_h, src_h, dst_h, out_h,
              acc, src_v, dst_v, rows_v, zero_v, sem):
    c = lax.axis_index("c")
    s = lax.axis_index("s")
    wid = c * NS + s

    zero16 = jnp.zeros((16,), _f32)

    def fill_zero(t, _):
        i = t // (D // 16)
        j = t % (D // 16)
        zero_v[i, pl.ds(j * 16, 16)] = zero16
        return 0

    lax.fori_loop(0, ZROWS * (D // 16), fill_zero, 0)

    base_r = s * ROWS_PT

    def zcopy(r, _):
        pltpu.sync_copy(zero_v, acc.at[pl.ds(base_r + r * ZROWS, ZROWS)])
        return 0

    lax.fori_loop(0, ROWS_PT // ZROWS, zcopy, 0)
    plsc.subcore_barrier()

    pltpu.sync_copy(src_h.at[wid], src_v)
    pltpu.sync_copy(dst_h.at[wid], dst_v)

    def body(j, _):
        pltpu.async_copy(table_h.at[src_v.at[j]], rows_v, sem).wait()
        pltpu.sync_copy(rows_v, acc.at[dst_v.at[j]], add=True)
        return 0

    lax.fori_loop(0, NCH, body, 0)
    plsc.subcore_barrier()

    pltpu.sync_copy(acc.at[pl.ds(base_r, ROWS_PT)],
                    out_h.at[c, pl.ds(base_r, ROWS_PT)])


def _bn(h, g, b):
    m = jnp.mean(h, axis=0, keepdims=True)
    v = jnp.mean((h - m) ** 2, axis=0, keepdims=True)
    return (h - m) / jnp.sqrt(v + 1e-5) * g + b


def _silu2(h):
    return h * (1.0 / (1.0 + jnp.exp(-h)))


def _tc_prep(dcnt, bcnt):
    def body(d_ref, b_ref, dinv_ref, binv_ref):
        d = d_ref[0, :N, :1] + d_ref[1, :N, :1]
        b = b_ref[0, :N, :1] + b_ref[1, :N, :1]
        dinv_ref[...] = jnp.broadcast_to(
            jnp.where(d > 0, 1.0 / d, 0.0), (N, DEG_D))
        binv_ref[...] = jnp.broadcast_to(
            jnp.where(b > 0, 1.0 / b, 0.0), (N, DEG_D))

    return pl.pallas_call(
        body,
        out_shape=[jax.ShapeDtypeStruct((N, DEG_D), _f32),
                   jax.ShapeDtypeStruct((N, DEG_D), _f32)],
    )(dcnt, bcnt)


def _split(a):
    hi = a.astype(jnp.bfloat16)
    lo = (a - hi.astype(_f32)).astype(jnp.bfloat16)
    return hi, lo


def _mm(a, w):
    return lax.dot_general(a, w, (((1,), (1,)), ((), ())),
                           preferred_element_type=_f32)


def _tc_scale(e_parts, binv16):
    def body(e_ref, binv_ref, o_ref):
        o_ref[...] = (e_ref[0, :N] + e_ref[1, :N]) * binv_ref[:, :1]

    return pl.pallas_call(
        body, out_shape=jax.ShapeDtypeStruct((N, D), _f32)
    )(e_parts, binv16)


def _tc_proj0(x, w0):
    def body(x_ref, w0_ref, o_ref):
        o_ref[0] = _mm(x_ref[...], w0_ref[0])

    return pl.pallas_call(
        body,
        grid=(4,),
        in_specs=[
            pl.BlockSpec((N, D), lambda br: (0, 0)),
            pl.BlockSpec((1, 64, D), lambda br: (br, 0, 0)),
        ],
        out_specs=pl.BlockSpec((1, N, 64), lambda br: (br, 0, 0)),
        out_shape=jax.ShapeDtypeStruct((4, N, 64), _f32),
    )(x, w0)


def _tc_scale2(n2a, n2b, dinv16):
    def body(na_ref, nb_ref, dinv_ref, o_ref):
        dinv = dinv_ref[:, :1]
        o_ref[0] = (na_ref[0, :N] + na_ref[1, :N]) * dinv
        o_ref[1] = (nb_ref[0, :N] + nb_ref[1, :N]) * dinv

    return pl.pallas_call(
        body, out_shape=jax.ShapeDtypeStruct((2, N, D), _f32)
    )(n2a, n2b, dinv16)


def _tc_branch(z12, b, g, be, w, dout):
    def body(z_ref, b_ref, g_ref, be_ref, w_ref, o_ref):
        br = pl.program_id(0)
        zf = z_ref[0]
        z = jnp.where(br % 2 == 0, zf[:, :64], zf[:, 64:]) + b_ref[0]
        h = _bn(z, g_ref[0], be_ref[0])
        h = _silu2(h)
        o_ref[0] = _mm(h, w_ref[0])

    return pl.pallas_call(
        body,
        grid=(4,),
        in_specs=[
            pl.BlockSpec((1, N, D), lambda br: (br // 2, 0, 0)),
            pl.BlockSpec((1, 1, 64), lambda br: (br, 0, 0)),
            pl.BlockSpec((1, 1, 64), lambda br: (br, 0, 0)),
            pl.BlockSpec((1, 1, 64), lambda br: (br, 0, 0)),
            pl.BlockSpec((1, dout, 64), lambda br: (br, 0, 0)),
        ],
        out_specs=pl.BlockSpec((1, N, dout), lambda br: (br, 0, 0)),
        out_shape=jax.ShapeDtypeStruct((4, N, dout), _f32),
    )(z12, b, g, be, w)


def _tc_final(n3, dinv16, b2, g2, be2, batch2d,
              p_w1, p_b1, p_g, p_be, p_w2, p_b2):
    def body(n_ref, dinv_ref, b2_ref, g2_ref, be2_ref, batch_ref,
             pw1_ref, pb1_ref, pg_ref, pbe_ref, pw2_ref, pb2_ref, o_ref):
        z2 = (n_ref[0, :N] + n_ref[1, :N]) * dinv_ref[:, :1]
        h = jnp.zeros((N, 32), _f32)
        for br in range(4):
            z = z2[:, br * 32:(br + 1) * 32] + b2_ref[br:br + 1, :]
            z = _bn(z, g2_ref[br:br + 1, :], be2_ref[br:br + 1, :])
            h = h + _silu2(z)
        gid = lax.broadcasted_iota(jnp.int32, (64, N), 0)
        onehot = (gid == batch_ref[...]).astype(_f32)
        h_hi, h_lo = _split(h)
        oh = onehot.astype(jnp.bfloat16)
        dn = (((1,), (0,)), ((), ()))
        sums = (lax.dot_general(oh, h_hi, dn, preferred_element_type=_f32)
                + lax.dot_general(oh, h_lo, dn, preferred_element_type=_f32))
        cnt = jnp.sum(onehot, axis=1, keepdims=True)
        pooled = sums / jnp.maximum(cnt, 1.0)
        z = _mm(pooled, pw1_ref[...]) + pb1_ref[...]
        z = _bn(z, pg_ref[...], pbe_ref[...])
        z = _silu2(z)
        o_ref[...] = _mm(z, pw2_ref[...]) + pb2_ref[...]

    return pl.pallas_call(
        body, out_shape=jax.ShapeDtypeStruct((64, 10), _f32)
    )(n3, dinv16, b2, g2, be2, batch2d, p_w1, p_b1, p_g, p_be, p_w2, p_b2)


def kernel(x, hyperedge_index, batch, params):
    node_rs = hyperedge_index[0].reshape(NW, NCH, CHUNK)
    edge_rs = hyperedge_index[1].reshape(NW, NCH, CHUNK)

    st = lambda fmt: jnp.stack([params[fmt.format(br)] for br in range(4)])
    st3 = lambda fmt: st(fmt)[:, None, :]
    w0, b0 = st("b{}_l0_W"), st3("b{}_l0_b")
    g0, be0 = st3("b{}_l0_g"), st3("b{}_l0_be")
    w1, b1 = st("b{}_l1_W"), st3("b{}_l1_b")
    g1c, be1 = st3("b{}_l1_g"), st3("b{}_l1_be")
    w2, b2 = st("b{}_l2_W"), st("b{}_l2_b")
    g2, be2 = st("b{}_l2_g"), st("b{}_l2_be")

    ones_in = jnp.ones((CHUNK, D), _f32)
    dcnt = _sc_count(ones_in, node_rs)
    bcnt = _sc_count(ones_in, edge_rs)
    dinv16, binv16 = _tc_prep(dcnt, bcnt)

    c0 = _tc_proj0(x, w0)
    c0a = jnp.concatenate([c0[0], c0[1]], axis=1)
    c0b = jnp.concatenate([c0[2], c0[3]], axis=1)
    e1a = _sc_phase(c0a, node_rs, edge_rs)
    e1b = _sc_phase(c0b, node_rs, edge_rs)
    v1a = _tc_scale(e1a, binv16)
    v1b = _tc_scale(e1b, binv16)
    n1a = _sc_phase(v1a, edge_rs, node_rs)
    n1b = _sc_phase(v1b, edge_rs, node_rs)
    z01 = _tc_scale2(n1a, n1b, dinv16)

    g1s = _tc_branch(z01, b0, g0, be0, w1, 64)
    g1a = jnp.concatenate([g1s[0], g1s[1]], axis=1)
    g1b = jnp.concatenate([g1s[2], g1s[3]], axis=1)
    e2a = _sc_phase(g1a, node_rs, edge_rs)
    e2b = _sc_phase(g1b, node_rs, edge_rs)
    v2a = _tc_scale(e2a, binv16)
    v2b = _tc_scale(e2b, binv16)
    n2a = _sc_phase(v2a, edge_rs, node_rs)
    n2b = _sc_phase(v2b, edge_rs, node_rs)
    z12 = _tc_scale2(n2a, n2b, dinv16)

    g2s = _tc_branch(z12, b1, g1c, be1, w2, 32)
    g2t = jnp.concatenate([g2s[0], g2s[1], g2s[2], g2s[3]], axis=1)
    e3 = _sc_phase(g2t, node_rs, edge_rs)
    v3 = _tc_scale(e3, binv16)
    n3 = _sc_phase(v3, edge_rs, node_rs)

    return _tc_final(
        n3, dinv16, b2, g2, be2, batch.reshape(1, N),
        params["p_W1"], params["p_b1"].reshape(1, 64),
        params["p_g"].reshape(1, 64), params["p_be"].reshape(1, 64),
        params["p_W2"], params["p_b2"].reshape(1, 10),
    )


# speedup vs baseline: 17.5041x; 1.4790x over previous
import functools

import jax
import jax.numpy as jnp
from jax import lax
from jax.experimental import pallas as pl
from jax.experimental.pallas import tpu as pltpu
from jax.experimental.pallas import tpu_sc as plsc

N = 10000
NP = 10240
D = 128
E_INC = 320000
NC, NS = 2, 16
NW = NC * NS
PER_TILE = E_INC // NW
CHUNK = 80
NCH = PER_TILE // CHUNK
ROWS_PT = NP // NS
SEC = 5
NCHS = NCH // SEC
DEG_D = 16

_MESH = plsc.VectorSubcoreMesh(
    core_axis_name="c", subcore_axis_name="s", num_cores=NC, num_subcores=NS
)

_f32 = jnp.float32


@functools.partial(
    pl.kernel,
    out_type=jax.ShapeDtypeStruct((NC, NP, D), _f32),
    mesh=_MESH,
    scratch_types=[
        pltpu.VMEM_SHARED((NP, D), _f32),
        pltpu.VMEM((NCH, CHUNK), jnp.int32),
        pltpu.VMEM((CHUNK, D), _f32),
        pltpu.SemaphoreType.DMA,
        pltpu.SemaphoreType.DMA,
    ],
)
def _sc_count(ones_h, dst_h, zeros_h, out_h, acc, dst_v, rows_v, semg, sems):
    c = lax.axis_index("c")
    s = lax.axis_index("s")
    wid = c * NS + s
    base_r = s * ROWS_PT

    pltpu.sync_copy(zeros_h.at[pl.ds(base_r, ROWS_PT)],
                    acc.at[pl.ds(base_r, ROWS_PT)])
    plsc.subcore_barrier()

    pltpu.sync_copy(dst_h.at[wid], dst_v)
    pltpu.async_copy(ones_h.at[pl.ds(0, CHUNK)], rows_v, semg).wait()

    def grp(g, _):
        for b in range(5):
            pltpu.async_copy(rows_v, acc.at[dst_v.at[g * 5 + b]], sems,
                             add=True)
        for b in range(5):
            pltpu.make_async_copy(ones_h.at[pl.ds(0, CHUNK)], rows_v,
                                  sems).wait()
        return 0

    lax.fori_loop(0, NCH // 5, grp, 0)
    plsc.subcore_barrier()
    pltpu.sync_copy(acc.at[pl.ds(base_r, ROWS_PT)],
                    out_h.at[c, pl.ds(base_r, ROWS_PT)])


@functools.partial(
    pl.kernel,
    out_type=jax.ShapeDtypeStruct((NC, NP, D), _f32),
    mesh=_MESH,
    scratch_types=[
        pltpu.VMEM_SHARED((NP, D), _f32),
        pltpu.VMEM((NCHS, CHUNK), jnp.int32),
        pltpu.VMEM((NCHS, CHUNK), jnp.int32),
        pltpu.VMEM((CHUNK, D), _f32),
        pltpu.VMEM((CHUNK, D), _f32),
        pltpu.SemaphoreType.DMA,
        pltpu.SemaphoreType.DMA,
    ],
)
def _sc_phase(table_h, src_h, dst_h, zeros_h, out_h,
              acc, src_v, dst_v, rows0, rows1, sem0, sem1):
    c = lax.axis_index("c")
    s = lax.axis_index("s")
    wid = c * NS + s
    base_r = s * ROWS_PT

    pltpu.sync_copy(zeros_h.at[pl.ds(base_r, ROWS_PT)],
                    acc.at[pl.ds(base_r, ROWS_PT)])
    plsc.subcore_barrier()

    def wait(buf, sem):
        pltpu.make_async_copy(table_h.at[pl.ds(0, CHUNK)], buf, sem).wait()

    def section(sec, _):
        pltpu.sync_copy(src_h.at[wid, sec], src_v)
        pltpu.sync_copy(dst_h.at[wid, sec], dst_v)
        pltpu.async_copy(table_h.at[src_v.at[0]], rows0, sem0)

        def pair(t, _):
            j0 = 2 * t
            pltpu.async_copy(table_h.at[src_v.at[j0 + 1]], rows1, sem1)
            wait(rows0, sem0)
            pltpu.sync_copy(rows0, acc.at[dst_v.at[j0]], add=True)
            pltpu.async_copy(table_h.at[src_v.at[j0 + 2]], rows0, sem0)
            wait(rows1, sem1)
            pltpu.sync_copy(rows1, acc.at[dst_v.at[j0 + 1]], add=True)
            return 0

        lax.fori_loop(0, (NCHS - 1) // 2, pair, 0)
        wait(rows0, sem0)
        pltpu.sync_copy(rows0, acc.at[dst_v.at[NCHS - 1]], add=True)
        return 0

    lax.fori_loop(0, SEC, section, 0)
    plsc.subcore_barrier()
    pltpu.sync_copy(acc.at[pl.ds(base_r, ROWS_PT)],
                    out_h.at[c, pl.ds(base_r, ROWS_PT)])


def _bn(h, g, b):
    m = jnp.mean(h, axis=0, keepdims=True)
    v = jnp.mean((h - m) ** 2, axis=0, keepdims=True)
    return (h - m) / jnp.sqrt(v + 1e-5) * g + b


def _silu2(h):
    return h * (1.0 / (1.0 + jnp.exp(-h)))


def _tc_prep(dcnt, bcnt):
    def body(d_ref, b_ref, dinv_ref, binv_ref):
        d = d_ref[0, :N, :1] + d_ref[1, :N, :1]
        b = b_ref[0, :N, :1] + b_ref[1, :N, :1]
        dinv_ref[...] = jnp.broadcast_to(
            jnp.where(d > 0, 1.0 / d, 0.0), (N, DEG_D))
        binv_ref[...] = jnp.broadcast_to(
            jnp.where(b > 0, 1.0 / b, 0.0), (N, DEG_D))

    return pl.pallas_call(
        body,
        out_shape=[jax.ShapeDtypeStruct((N, DEG_D), _f32),
                   jax.ShapeDtypeStruct((N, DEG_D), _f32)],
    )(dcnt, bcnt)


def _split(a):
    hi = a.astype(jnp.bfloat16)
    lo = (a - hi.astype(_f32)).astype(jnp.bfloat16)
    return hi, lo


def _mm(a, w):
    return lax.dot_general(a, w, (((1,), (1,)), ((), ())),
                           preferred_element_type=_f32)


def _tc_scale(e_parts, binv16):
    def body(e_ref, binv_ref, o_ref):
        o_ref[...] = (e_ref[0, :N] + e_ref[1, :N]) * binv_ref[:, :1]

    return pl.pallas_call(
        body, out_shape=jax.ShapeDtypeStruct((N, D), _f32)
    )(e_parts, binv16)


def _tc_proj0(x, w0):
    def body(x_ref, w0_ref, o_ref):
        o_ref[0] = _mm(x_ref[...], w0_ref[0])

    return pl.pallas_call(
        body,
        grid=(4,),
        in_specs=[
            pl.BlockSpec((N, D), lambda br: (0, 0)),
            pl.BlockSpec((1, 64, D), lambda br: (br, 0, 0)),
        ],
        out_specs=pl.BlockSpec((1, N, 64), lambda br: (br, 0, 0)),
        out_shape=jax.ShapeDtypeStruct((4, N, 64), _f32),
    )(x, w0)


def _tc_scale2(n2a, n2b, dinv16):
    def body(na_ref, nb_ref, dinv_ref, o_ref):
        dinv = dinv_ref[:, :1]
        o_ref[0] = (na_ref[0, :N] + na_ref[1, :N]) * dinv
        o_ref[1] = (nb_ref[0, :N] + nb_ref[1, :N]) * dinv

    return pl.pallas_call(
        body, out_shape=jax.ShapeDtypeStruct((2, N, D), _f32)
    )(n2a, n2b, dinv16)


def _tc_branch(z12, b, g, be, w, dout):
    def body(z_ref, b_ref, g_ref, be_ref, w_ref, o_ref):
        br = pl.program_id(0)
        zf = z_ref[0]
        z = jnp.where(br % 2 == 0, zf[:, :64], zf[:, 64:]) + b_ref[0]
        h = _bn(z, g_ref[0], be_ref[0])
        h = _silu2(h)
        o_ref[0] = _mm(h, w_ref[0])

    return pl.pallas_call(
        body,
        grid=(4,),
        in_specs=[
            pl.BlockSpec((1, N, D), lambda br: (br // 2, 0, 0)),
            pl.BlockSpec((1, 1, 64), lambda br: (br, 0, 0)),
            pl.BlockSpec((1, 1, 64), lambda br: (br, 0, 0)),
            pl.BlockSpec((1, 1, 64), lambda br: (br, 0, 0)),
            pl.BlockSpec((1, dout, 64), lambda br: (br, 0, 0)),
        ],
        out_specs=pl.BlockSpec((1, N, dout), lambda br: (br, 0, 0)),
        out_shape=jax.ShapeDtypeStruct((4, N, dout), _f32),
    )(z12, b, g, be, w)


def _tc_final(n3, dinv16, b2, g2, be2, batch2d,
              p_w1, p_b1, p_g, p_be, p_w2, p_b2):
    def body(n_ref, dinv_ref, b2_ref, g2_ref, be2_ref, batch_ref,
             pw1_ref, pb1_ref, pg_ref, pbe_ref, pw2_ref, pb2_ref, o_ref):
        z2 = (n_ref[0, :N] + n_ref[1, :N]) * dinv_ref[:, :1]
        h = jnp.zeros((N, 32), _f32)
        for br in range(4):
            z = z2[:, br * 32:(br + 1) * 32] + b2_ref[br:br + 1, :]
            z = _bn(z, g2_ref[br:br + 1, :], be2_ref[br:br + 1, :])
            h = h + _silu2(z)
        gid = lax.broadcasted_iota(jnp.int32, (64, N), 0)
        onehot = (gid == batch_ref[...]).astype(_f32)
        h_hi, h_lo = _split(h)
        oh = onehot.astype(jnp.bfloat16)
        dn = (((1,), (0,)), ((), ()))
        sums = (lax.dot_general(oh, h_hi, dn, preferred_element_type=_f32)
                + lax.dot_general(oh, h_lo, dn, preferred_element_type=_f32))
        cnt = jnp.sum(onehot, axis=1, keepdims=True)
        pooled = sums / jnp.maximum(cnt, 1.0)
        z = _mm(pooled, pw1_ref[...]) + pb1_ref[...]
        z = _bn(z, pg_ref[...], pbe_ref[...])
        z = _silu2(z)
        o_ref[...] = _mm(z, pw2_ref[...]) + pb2_ref[...]

    return pl.pallas_call(
        body, out_shape=jax.ShapeDtypeStruct((64, 10), _f32)
    )(n3, dinv16, b2, g2, be2, batch2d, p_w1, p_b1, p_g, p_be, p_w2, p_b2)


def kernel(x, hyperedge_index, batch, params):
    node3 = hyperedge_index[0].reshape(NW, NCH, CHUNK)
    edge3 = hyperedge_index[1].reshape(NW, NCH, CHUNK)
    node4 = hyperedge_index[0].reshape(NW, SEC, NCHS, CHUNK)
    edge4 = hyperedge_index[1].reshape(NW, SEC, NCHS, CHUNK)

    st = lambda fmt: jnp.stack([params[fmt.format(br)] for br in range(4)])
    st3 = lambda fmt: st(fmt)[:, None, :]
    w0, b0 = st("b{}_l0_W"), st3("b{}_l0_b")
    g0, be0 = st3("b{}_l0_g"), st3("b{}_l0_be")
    w1, b1 = st("b{}_l1_W"), st3("b{}_l1_b")
    g1c, be1 = st3("b{}_l1_g"), st3("b{}_l1_be")
    w2 = st("b{}_l2_W")
    b2, g2, be2 = st("b{}_l2_b"), st("b{}_l2_g"), st("b{}_l2_be")

    ones_in = jnp.ones((CHUNK, D), _f32)
    zeros_in = jnp.zeros((NP, D), _f32)
    dcnt = _sc_count(ones_in, node3, zeros_in)
    bcnt = _sc_count(ones_in, edge3, zeros_in)
    dinv16, binv16 = _tc_prep(dcnt, bcnt)

    c0 = _tc_proj0(x, w0)
    c0a = jnp.concatenate([c0[0], c0[1]], axis=1)
    c0b = jnp.concatenate([c0[2], c0[3]], axis=1)
    e1a = _sc_phase(c0a, node4, edge4, zeros_in)
    e1b = _sc_phase(c0b, node4, edge4, zeros_in)
    v1a = _tc_scale(e1a, binv16)
    v1b = _tc_scale(e1b, binv16)
    n1a = _sc_phase(v1a, edge4, node4, zeros_in)
    n1b = _sc_phase(v1b, edge4, node4, zeros_in)
    z01 = _tc_scale2(n1a, n1b, dinv16)

    g1s = _tc_branch(z01, b0, g0, be0, w1, 64)
    g1a = jnp.concatenate([g1s[0], g1s[1]], axis=1)
    g1b = jnp.concatenate([g1s[2], g1s[3]], axis=1)
    e2a = _sc_phase(g1a, node4, edge4, zeros_in)
    e2b = _sc_phase(g1b, node4, edge4, zeros_in)
    v2a = _tc_scale(e2a, binv16)
    v2b = _tc_scale(e2b, binv16)
    n2a = _sc_phase(v2a, edge4, node4, zeros_in)
    n2b = _sc_phase(v2b, edge4, node4, zeros_in)
    z12 = _tc_scale2(n2a, n2b, dinv16)

    g2s = _tc_branch(z12, b1, g1c, be1, w2, 32)
    g2t = jnp.concatenate([g2s[0], g2s[1], g2s[2], g2s[3]], axis=1)
    e3 = _sc_phase(g2t, node4, edge4, zeros_in)
    v3 = _tc_scale(e3, binv16)
    n3 = _sc_phase(v3, edge4, node4, zeros_in)

    return _tc_final(
        n3, dinv16, b2, g2, be2, batch.reshape(1, N),
        params["p_W1"], params["p_b1"].reshape(1, 64),
        params["p_g"].reshape(1, 64), params["p_be"].reshape(1, 64),
        params["p_W2"], params["p_b2"].reshape(1, 10),
    )
